# Initial kernel scaffold; baseline (speedup 1.0000x reference)
import jax
import jax.numpy as jnp
from jax.experimental import pallas as pl


def kernel(z, embedding):
    raise NotImplementedError("write your pallas kernel here")


# speedup vs baseline: 1.2137x; 1.2137x over previous
import jax
import jax.numpy as jnp
from jax.experimental import pallas as pl
from jax.experimental.pallas import tpu as pltpu

_T = 256


def _vq_body(zr_ref, emb_ref, embT_ref, zq_ref, idx_ref, perp_ref, md_ref,
             counts_ref, acc_ref, *, nb, nj, n_tokens, k):
    bi = pl.program_id(0)
    ji = pl.program_id(1)

    @pl.when((bi == 0) & (ji == 0))
    def _init():
        counts_ref[...] = jnp.zeros_like(counts_ref)
        acc_ref[0, 0] = 0.0

    zt = zr_ref[0]
    e = emb_ref[...]
    scores = jnp.dot(e, zt, preferred_element_type=jnp.float32)
    e2 = jnp.sum(e * e, axis=1, keepdims=True)
    z2 = jnp.sum(zt * zt, axis=0, keepdims=True)
    dist = (z2 - 2.0 * scores) + e2

    m = jnp.min(dist, axis=0, keepdims=True)
    kiota = jax.lax.broadcasted_iota(jnp.int32, dist.shape, 0)
    idx = jnp.min(jnp.where(dist == m, kiota, k), axis=0)
    idx_ref[0, 0, :] = idx

    onehot = (kiota == idx[None, :]).astype(jnp.float32)
    zq = jnp.dot(embT_ref[...], onehot, preferred_element_type=jnp.float32)
    zq_ref[0] = zq

    t = zt.shape[1]
    for g in range(t // 128):
        counts_ref[...] += onehot[:, g * 128:(g + 1) * 128]
    acc_ref[0, 0] += jnp.sum(dist)

    @pl.when((bi == nb - 1) & (ji == nj - 1))
    def _fini():
        cnt = jnp.sum(counts_ref[...], axis=1, keepdims=True)
        e_mean = cnt * (1.0 / n_tokens)
        ent = jnp.sum(e_mean * jnp.log(e_mean + 1e-10))
        perp_ref[0, 0] = jnp.exp(-ent)
        md_ref[0, 0] = acc_ref[0, 0] * (1.0 / (n_tokens * k))


def kernel(z, embedding):
    b, c, h, w, d = z.shape
    k = embedding.shape[0]
    hwd = h * w * d
    n_tokens = b * hwd
    t = _T
    nj = hwd // t

    zr = z.reshape(b, c, hwd)
    embT = embedding.T

    grid = (b, nj)
    out_shapes = (
        jax.ShapeDtypeStruct((b, c, hwd), jnp.float32),
        jax.ShapeDtypeStruct((b * nj, 1, t), jnp.int32),
        jax.ShapeDtypeStruct((1, 1), jnp.float32),
        jax.ShapeDtypeStruct((1, 1), jnp.float32),
    )
    in_specs = [
        pl.BlockSpec((1, c, t), lambda bi, ji: (bi, 0, ji)),
        pl.BlockSpec((k, c), lambda bi, ji: (0, 0)),
        pl.BlockSpec((c, k), lambda bi, ji: (0, 0)),
    ]
    out_specs = (
        pl.BlockSpec((1, c, t), lambda bi, ji: (bi, 0, ji)),
        pl.BlockSpec((1, 1, t), lambda bi, ji: (bi * nj + ji, 0, 0)),
        pl.BlockSpec(memory_space=pltpu.SMEM),
        pl.BlockSpec(memory_space=pltpu.SMEM),
    )
    import functools
    body = functools.partial(_vq_body, nb=b, nj=nj, n_tokens=n_tokens, k=k)
    zq, idx, perp, md = pl.pallas_call(
        body,
        grid=grid,
        in_specs=in_specs,
        out_specs=out_specs,
        out_shape=out_shapes,
        scratch_shapes=[
            pltpu.VMEM((k, 128), jnp.float32),
            pltpu.SMEM((1, 1), jnp.float32),
        ],
    )(zr, embedding, embT)

    z_q = zq.reshape(b, c, h, w, d)
    indices = idx.reshape(b, h, w, d)
    loss = jnp.zeros((), z.dtype)
    return (z_q, loss, perp[0, 0], indices, md[0, 0])


# speedup vs baseline: 1.2184x; 1.0039x over previous
import functools

import jax
import jax.numpy as jnp
from jax import lax
from jax.experimental import pallas as pl
from jax.experimental.pallas import tpu as pltpu
from jax.experimental.pallas import tpu_sc as plsc

_T = 512


def _vq_body(zr_ref, eneg2_ref, e2_ref, emb_ref, idx_ref, perp_ref, md_ref,
             counts_ref, szacc_ref, z2acc_ref, *, nb, nj, n_tokens, k):
    bi = pl.program_id(0)
    ji = pl.program_id(1)

    @pl.when((bi == 0) & (ji == 0))
    def _init():
        counts_ref[...] = jnp.zeros_like(counts_ref)
        szacc_ref[...] = jnp.zeros_like(szacc_ref)
        z2acc_ref[0, 0] = 0.0

    zt = zr_ref[0]
    c, t = zt.shape
    v = jnp.dot(eneg2_ref[...], zt, preferred_element_type=jnp.float32)
    v = v + e2_ref[...]

    m = jnp.min(v, axis=0, keepdims=True)
    kiota = jax.lax.broadcasted_iota(jnp.int32, v.shape, 0)
    idx = jnp.min(jnp.where(v == m, kiota, k), axis=0)
    idx_ref[0, 0, :] = idx

    onehot = (kiota == idx[None, :]).astype(jnp.float32)
    for g in range(t // 128):
        counts_ref[...] += onehot[:, g * 128:(g + 1) * 128]
        szacc_ref[...] += zt[:, g * 128:(g + 1) * 128]
    z2acc_ref[0, 0] += jnp.sum(zt * zt)

    @pl.when((bi == nb - 1) & (ji == nj - 1))
    def _fini():
        cnt = jnp.sum(counts_ref[...], axis=1, keepdims=True)
        e_mean = cnt * (1.0 / n_tokens)
        ent = jnp.sum(e_mean * jnp.log(e_mean + 1e-10))
        perp_ref[0, 0] = jnp.exp(-ent)
        e = emb_ref[...]
        sum_e2 = jnp.sum(e * e)
        sz = jnp.sum(szacc_ref[...], axis=1, keepdims=True)
        se = jnp.sum(e, axis=0, keepdims=True)
        cross = jnp.sum(se * sz.T)
        tot = (k * z2acc_ref[0, 0] + n_tokens * sum_e2 - 2.0 * cross)
        md_ref[0, 0] = tot * (1.0 / (n_tokens * k))


def _tc_quantize(zr, e_neg2, e2, embedding):
    b, c, hwd = zr.shape
    k = embedding.shape[0]
    n_tokens = b * hwd
    t = _T
    nj = hwd // t

    grid = (b, nj)
    out_shapes = (
        jax.ShapeDtypeStruct((b * nj, 1, t), jnp.int32),
        jax.ShapeDtypeStruct((1, 1), jnp.float32),
        jax.ShapeDtypeStruct((1, 1), jnp.float32),
    )
    in_specs = [
        pl.BlockSpec((1, c, t), lambda bi, ji: (bi, 0, ji)),
        pl.BlockSpec((k, c), lambda bi, ji: (0, 0)),
        pl.BlockSpec((k, 1), lambda bi, ji: (0, 0)),
        pl.BlockSpec((k, c), lambda bi, ji: (0, 0)),
    ]
    out_specs = (
        pl.BlockSpec((1, 1, t), lambda bi, ji: (bi * nj + ji, 0, 0)),
        pl.BlockSpec(memory_space=pltpu.SMEM),
        pl.BlockSpec(memory_space=pltpu.SMEM),
    )
    body = functools.partial(_vq_body, nb=b, nj=nj, n_tokens=n_tokens, k=k)
    return pl.pallas_call(
        body,
        grid=grid,
        in_specs=in_specs,
        out_specs=out_specs,
        out_shape=out_shapes,
        scratch_shapes=[
            pltpu.VMEM((k, 128), jnp.float32),
            pltpu.VMEM((c, 128), jnp.float32),
            pltpu.SMEM((1, 1), jnp.float32),
        ],
    )(zr, e_neg2, e2, embedding)


def _sc_gather(embT, idx_flat, b, c, hwd):
    n_tokens = b * hwd
    ncg = 8
    cpg = c // ncg
    ntr = 32 // ncg
    tpr = n_tokens // ntr
    mesh = plsc.VectorSubcoreMesh(core_axis_name="c", subcore_axis_name="s")

    @functools.partial(
        pl.kernel, mesh=mesh,
        compiler_params=pltpu.CompilerParams(needs_layout_passes=False),
        out_type=jax.ShapeDtypeStruct((b, c, hwd), jnp.float32),
        scratch_types=[
            pltpu.VMEM((cpg * 1024,), jnp.float32),
            pltpu.VMEM((tpr,), jnp.int32),
            pltpu.VMEM((cpg, tpr), jnp.float32),
        ],
    )
    def k(embT_hbm, idx_hbm, out_hbm, embt_v, idx_v, out_v):
        wid = lax.axis_index("s") * 2 + lax.axis_index("c")
        cg = wid % ncg
        tr = wid // ncg
        pltpu.sync_copy(embT_hbm.at[pl.ds(cg * cpg * 1024, cpg * 1024)],
                        embt_v)
        pltpu.sync_copy(idx_hbm.at[pl.ds(tr * tpr, tpr)], idx_v)

        def body(g, carry):
            iv = idx_v[pl.ds(g * 16, 16)]
            for ch in range(cpg):
                out_v[ch, pl.ds(g * 16, 16)] = plsc.load_gather(
                    embt_v, [iv + (ch * 1024)])
            return carry

        lax.fori_loop(0, tpr // 16, body, 0)
        pltpu.sync_copy(out_v, out_hbm.at[tr, pl.ds(cg * cpg, cpg), :])

    return k(embT, idx_flat)


def kernel(z, embedding):
    b, c, h, w, d = z.shape
    k = embedding.shape[0]
    hwd = h * w * d

    zr = z.reshape(b, c, hwd)
    e2 = jnp.sum(embedding * embedding, axis=1, keepdims=True)
    e_neg2 = -2.0 * embedding
    embT = embedding.T.reshape(c * k)

    idx, perp, md = _tc_quantize(zr, e_neg2, e2, embedding)
    idx_flat = idx.reshape(b * hwd)
    zq = _sc_gather(embT, idx_flat, b, c, hwd)

    z_q = zq.reshape(b, c, h, w, d)
    indices = idx.reshape(b, h, w, d)
    loss = jnp.zeros((), z.dtype)
    return (z_q, loss, perp[0, 0], indices, md[0, 0])


# speedup vs baseline: 1.4772x; 1.2124x over previous
import functools

import jax
import jax.numpy as jnp
from jax import lax
from jax.experimental import pallas as pl
from jax.experimental.pallas import tpu as pltpu
from jax.experimental.pallas import tpu_sc as plsc

_T = 512
_CPAD = 72


def _vq_body(zr_ref, eaug_ref, emb_ref, idx_ref, perp_ref, md_ref,
             counts_ref, szacc_ref, z2acc_ref, *, nb, nj, n_tokens, k):
    bi = pl.program_id(0)
    ji = pl.program_id(1)

    @pl.when((bi == 0) & (ji == 0))
    def _init():
        counts_ref[...] = jnp.zeros_like(counts_ref)
        szacc_ref[...] = jnp.zeros_like(szacc_ref)
        z2acc_ref[0, 0] = 0.0

    zt = zr_ref[0]
    c, t = zt.shape
    zaug = jnp.concatenate(
        [zt, jnp.ones((3, t), jnp.float32),
         jnp.zeros((_CPAD - c - 3, t), jnp.float32)], axis=0)
    v = jnp.dot(eaug_ref[...], zaug, preferred_element_type=jnp.float32)

    idx = jnp.argmin(v, axis=0).astype(jnp.int32)
    idx_ref[0, 0, :] = idx

    kiota = jax.lax.broadcasted_iota(jnp.int32, (k, 128), 0)
    for g in range(t // 128):
        lo, hi = g * 128, (g + 1) * 128
        counts_ref[...] += (kiota == idx[lo:hi][None, :]).astype(jnp.float32)
        szacc_ref[...] += zt[:, lo:hi]
    z2acc_ref[0, 0] += jnp.sum(zt * zt)

    @pl.when((bi == nb - 1) & (ji == nj - 1))
    def _fini():
        cnt = jnp.sum(counts_ref[...], axis=1, keepdims=True)
        e_mean = cnt * (1.0 / n_tokens)
        ent = jnp.sum(e_mean * jnp.log(e_mean + 1e-10))
        perp_ref[0, 0] = jnp.exp(-ent)
        e = emb_ref[...]
        sum_e2 = jnp.sum(e * e)
        sz = jnp.sum(szacc_ref[...], axis=1, keepdims=True)
        se = jnp.sum(e, axis=0, keepdims=True)
        cross = jnp.sum(se * sz.T)
        tot = (k * z2acc_ref[0, 0] + n_tokens * sum_e2 - 2.0 * cross)
        md_ref[0, 0] = tot * (1.0 / (n_tokens * k))


def _tc_quantize(zr, e_aug, embedding):
    b, c, hwd = zr.shape
    k = embedding.shape[0]
    n_tokens = b * hwd
    t = _T
    nj = hwd // t

    grid = (b, nj)
    out_shapes = (
        jax.ShapeDtypeStruct((b * nj, 1, t), jnp.int32),
        jax.ShapeDtypeStruct((1, 1), jnp.float32),
        jax.ShapeDtypeStruct((1, 1), jnp.float32),
    )
    in_specs = [
        pl.BlockSpec((1, c, t), lambda bi, ji: (bi, 0, ji)),
        pl.BlockSpec((k, _CPAD), lambda bi, ji: (0, 0)),
        pl.BlockSpec((k, c), lambda bi, ji: (0, 0)),
    ]
    out_specs = (
        pl.BlockSpec((1, 1, t), lambda bi, ji: (bi * nj + ji, 0, 0)),
        pl.BlockSpec(memory_space=pltpu.SMEM),
        pl.BlockSpec(memory_space=pltpu.SMEM),
    )
    body = functools.partial(_vq_body, nb=b, nj=nj, n_tokens=n_tokens, k=k)
    return pl.pallas_call(
        body,
        grid=grid,
        in_specs=in_specs,
        out_specs=out_specs,
        out_shape=out_shapes,
        scratch_shapes=[
            pltpu.VMEM((k, 128), jnp.float32),
            pltpu.VMEM((c, 128), jnp.float32),
            pltpu.SMEM((1, 1), jnp.float32),
        ],
    )(zr, e_aug, embedding)


def _sc_gather(embT, idx_flat, b, c, hwd):
    n_tokens = b * hwd
    ncg = 8
    cpg = c // ncg
    ntr = 32 // ncg
    tpr = n_tokens // ntr
    mesh = plsc.VectorSubcoreMesh(core_axis_name="c", subcore_axis_name="s")

    @functools.partial(
        pl.kernel, mesh=mesh,
        compiler_params=pltpu.CompilerParams(needs_layout_passes=False),
        out_type=jax.ShapeDtypeStruct((b, c, hwd), jnp.float32),
        scratch_types=[
            pltpu.VMEM((cpg * 1024,), jnp.float32),
            pltpu.VMEM((tpr,), jnp.int32),
            pltpu.VMEM((cpg, tpr), jnp.float32),
        ],
    )
    def k(embT_hbm, idx_hbm, out_hbm, embt_v, idx_v, out_v):
        wid = lax.axis_index("s") * 2 + lax.axis_index("c")
        cg = wid % ncg
        tr = wid // ncg
        pltpu.sync_copy(embT_hbm.at[pl.ds(cg * cpg * 1024, cpg * 1024)],
                        embt_v)
        pltpu.sync_copy(idx_hbm.at[pl.ds(tr * tpr, tpr)], idx_v)

        def body(g, carry):
            iv = idx_v[pl.ds(g * 16, 16)]
            for ch in range(cpg):
                out_v[ch, pl.ds(g * 16, 16)] = plsc.load_gather(
                    embt_v, [iv + (ch * 1024)])
            return carry

        lax.fori_loop(0, tpr // 16, body, 0)
        pltpu.sync_copy(out_v, out_hbm.at[tr, pl.ds(cg * cpg, cpg), :])

    return k(embT, idx_flat)


def kernel(z, embedding):
    b, c, h, w, d = z.shape
    k = embedding.shape[0]
    hwd = h * w * d

    zr = z.reshape(b, c, hwd)
    e2 = jnp.sum(embedding * embedding, axis=1, keepdims=True)
    e2_hi = e2.astype(jnp.bfloat16).astype(jnp.float32)
    r1 = e2 - e2_hi
    e2_mid = r1.astype(jnp.bfloat16).astype(jnp.float32)
    r2 = r1 - e2_mid
    e2_lo = r2.astype(jnp.bfloat16).astype(jnp.float32)
    e_aug = jnp.concatenate(
        [-2.0 * embedding, e2_hi, e2_mid, e2_lo,
         jnp.zeros((k, _CPAD - c - 3), jnp.float32)], axis=1)
    embT = embedding.T.reshape(c * k)

    idx, perp, md = _tc_quantize(zr, e_aug, embedding)
    idx_flat = idx.reshape(b * hwd)
    zq = _sc_gather(embT, idx_flat, b, c, hwd)

    z_q = zq.reshape(b, c, h, w, d)
    indices = idx.reshape(b, h, w, d)
    loss = jnp.zeros((), z.dtype)
    return (z_q, loss, perp[0, 0], indices, md[0, 0])
